# Initial kernel scaffold; baseline (speedup 1.0000x reference)
import jax
import jax.numpy as jnp
from jax.experimental import pallas as pl


def kernel(imgs, rots, trans, intrins, points, backbone_w, backbone_b, bn1_g, bn1_b, depth_w, depth_b, pfn_w, pfn_b, pfn_bn_g, pfn_bn_b, fuser_w1, fbn1_g, fbn1_b, fuser_w2, fbn2_g, fbn2_b):
    raise NotImplementedError("write your pallas kernel here")


# speedup vs baseline: 2.3872x; 2.3872x over previous
import functools

import jax
import jax.numpy as jnp
from jax import lax
from jax.experimental import pallas as pl
from jax.experimental.pallas import tpu as pltpu
from jax.experimental.pallas import tpu_sc as plsc

B = 2; N = 6; NP = 100000
IH = 256; IW = 704; FH = 16; FW = 44
D = 59; CAM_C = 256; OUT_C = 80; LID_C = 32; FUSE_C = 128
DX = jnp.array([0.8, 0.8, 20.0], dtype=jnp.float32)
BX = jnp.array([-50.8, -50.8, 0.0], dtype=jnp.float32)
NX = (128, 128, 1)

TOTAL = B * NX[0] * NX[1] * NX[2]
GRID = TOTAL + 128
TRASH0 = TOTAL + 32
NPRIME = B * N * D * FH * FW
NW = 32
NCAM_W = 15616
NPRIME_PAD = NCAM_W * NW
RC_CHUNK = NCAM_W // 2
NPIX = B * N * FH * FW
PIX_STRIDE = D * FH * FW
HW = FH * FW
NP_PAD = 200704
NLID_W = NP_PAD // NW
RL_CHUNK = NLID_W
CH = 256

def _conv2d(x, w, b=None, stride=1, pad=0):
    y = lax.conv_general_dilated(
        x, w, (stride, stride), [(pad, pad), (pad, pad)],
        dimension_numbers=('NCHW', 'OIHW', 'NCHW'))
    if b is not None:
        y = y + b[None, :, None, None]
    return y


def _bn2d(x, g, b, eps=1e-5):
    m = x.mean(axis=(0, 2, 3), keepdims=True)
    v = x.var(axis=(0, 2, 3), keepdims=True)
    return (x - m) / jnp.sqrt(v + eps) * g[None, :, None, None] + b[None, :, None, None]


def _bn1d(x, g, b, eps=1e-5):
    m = x.mean(axis=0, keepdims=True)
    v = x.var(axis=0, keepdims=True)
    return (x - m) / jnp.sqrt(v + eps) * g[None, :] + b[None, :]


def _make_frustum():
    ds = jnp.broadcast_to(jnp.arange(1.0, 60.0, 1.0, dtype=jnp.float32).reshape(-1, 1, 1), (D, FH, FW))
    xs = jnp.broadcast_to(jnp.linspace(0.0, IW - 1.0, FW, dtype=jnp.float32).reshape(1, 1, FW), (D, FH, FW))
    ys = jnp.broadcast_to(jnp.linspace(0.0, IH - 1.0, FH, dtype=jnp.float32).reshape(1, FH, 1), (D, FH, FW))
    return jnp.stack((xs, ys, ds), -1)


def _voxel_ranks(xyz, batch_ix):
    coords = ((xyz - (BX - DX / 2.0)) / DX).astype(jnp.int32)
    kept = ((coords[:, 0] >= 0) & (coords[:, 0] < NX[0])
            & (coords[:, 1] >= 0) & (coords[:, 1] < NX[1])
            & (coords[:, 2] >= 0) & (coords[:, 2] < NX[2]))
    ranks = (batch_ix * (NX[0] * NX[1] * NX[2]) + coords[:, 0]
             + coords[:, 1] * NX[0] + coords[:, 2] * NX[0] * NX[1])
    return jnp.where(kept, ranks, TOTAL).astype(jnp.int32)


@functools.cache
def _build_sc_kernels():
  mesh = plsc.VectorSubcoreMesh(
      core_axis_name="c", subcore_axis_name="s", num_cores=2, num_subcores=16)

  @functools.partial(
    pl.kernel,
    out_type=(jax.ShapeDtypeStruct((NW, GRID), jnp.int32),
              jax.ShapeDtypeStruct((NW, GRID), jnp.float32)),
    mesh=mesh,
    compiler_params=pltpu.CompilerParams(needs_layout_passes=False),
    scratch_types=[
        pltpu.VMEM((GRID,), jnp.int32),
        pltpu.VMEM((GRID,), jnp.float32),
        pltpu.VMEM((NCAM_W,), jnp.float32),
        pltpu.VMEM((RC_CHUNK,), jnp.int32),
    ],
)
  def _pool_scatter_cam(ranks_c_hbm, depth_hbm, wpix_hbm, wdep_hbm,
                        wpix_v, wdep_v, dep_v, rc_v):
      wid = lax.axis_index("s") * 2 + lax.axis_index("c")
      iota = lax.iota(jnp.int32, 16)
      shift_idx = jnp.minimum(iota + 1, 15)
      lane15 = iota == 15
      neg1 = jnp.full((16,), -1, jnp.int32)
      fzero = jnp.zeros((16,), jnp.float32)

      def init_body(i, _):
          wpix_v[pl.ds(i * 16, 16)] = neg1
          wdep_v[pl.ds(i * 16, 16)] = fzero
          return 0
      lax.fori_loop(0, GRID // 16, init_body, 0)

      cam_base = pl.multiple_of(wid * NCAM_W, 128)
      pltpu.sync_copy(depth_hbm.at[pl.ds(cam_base, NCAM_W)], dep_v)

      for cc in range(NCAM_W // RC_CHUNK):
          pltpu.sync_copy(ranks_c_hbm.at[pl.ds(pl.multiple_of(cam_base + cc * RC_CHUNK, 128), RC_CHUNK)], rc_v)

          def cam_body(j, _):
              r = rc_v[pl.ds(j * 16, 16)]
              pidx = (cam_base + cc * RC_CHUNK + j * 16) + iota
              k2 = (r << 4) | iota
              ks, vs = plsc.sort_key_val(k2, pidx)
              rs = lax.shift_right_logical(ks, 4)
              shifted = jnp.take_along_axis(rs, shift_idx, axis=0)
              keep = (rs != shifted) | lane15
              tidx = jnp.where(keep, rs, TRASH0 + iota)
              dval = plsc.load_gather(dep_v, [vs - cam_base])
              camv = vs // PIX_STRIDE
              rem = vs - camv * PIX_STRIDE
              pix = camv * HW + (rem - (rem // HW) * HW)
              plsc.store_scatter(wpix_v, [tidx], pix)
              plsc.store_scatter(wdep_v, [tidx], dval)
              return 0
          lax.fori_loop(0, RC_CHUNK // 16, cam_body, 0)

      pltpu.sync_copy(wpix_v, wpix_hbm.at[wid])
      pltpu.sync_copy(wdep_v, wdep_hbm.at[wid])

  @functools.partial(
    pl.kernel,
    out_type=jax.ShapeDtypeStruct((NW, GRID), jnp.int32),
    mesh=mesh,
    compiler_params=pltpu.CompilerParams(needs_layout_passes=False),
    scratch_types=[
        pltpu.VMEM((GRID,), jnp.int32),
        pltpu.VMEM((NLID_W,), jnp.int32),
    ],
)
  def _pool_scatter_lid(ranks_l_hbm, wlid_hbm, wlid_v, rl_v):
      wid = lax.axis_index("s") * 2 + lax.axis_index("c")
      iota = lax.iota(jnp.int32, 16)
      shift_idx = jnp.minimum(iota + 1, 15)
      lane15 = iota == 15
      neg1 = jnp.full((16,), -1, jnp.int32)
      lid_base = pl.multiple_of(wid * NLID_W, 128)

      def init_body(i, _):
          wlid_v[pl.ds(i * 16, 16)] = neg1
          return 0
      lax.fori_loop(0, GRID // 16, init_body, 0)
      pltpu.sync_copy(ranks_l_hbm.at[pl.ds(lid_base, NLID_W)], rl_v)

      def lid_body(j, _):
          r = rl_v[pl.ds(j * 16, 16)]
          pidx = lid_base + j * 16 + iota
          k2 = (r << 4) | iota
          ks, vs = plsc.sort_key_val(k2, pidx)
          rs = lax.shift_right_logical(ks, 4)
          shifted = jnp.take_along_axis(rs, shift_idx, axis=0)
          keep = (rs != shifted) | lane15
          tidx = jnp.where(keep, rs, TRASH0 + iota)
          plsc.store_scatter(wlid_v, [tidx], vs)
          return 0
      lax.fori_loop(0, NLID_W // 16, lid_body, 0)
      pltpu.sync_copy(wlid_v, wlid_hbm.at[wid])

  @functools.partial(
      pl.kernel,
      out_type=(jax.ShapeDtypeStruct((TOTAL, 128), jnp.float32),
                jax.ShapeDtypeStruct((TOTAL, LID_C), jnp.float32)),
      mesh=mesh,
      compiler_params=pltpu.CompilerParams(needs_layout_passes=False),
      scratch_types=[
          pltpu.VMEM((NW, CH), jnp.int32),
          pltpu.VMEM((NW, CH), jnp.float32),
          pltpu.VMEM((NW, CH), jnp.int32),
          pltpu.VMEM((CH,), jnp.int32),
          pltpu.VMEM((CH,), jnp.float32),
          pltpu.VMEM((CH,), jnp.int32),
          pltpu.VMEM((CH,), jnp.int32),
          pltpu.VMEM((CH, 128), jnp.float32),
          pltpu.VMEM((CH, 128), jnp.float32),
          pltpu.VMEM((CH, LID_C), jnp.float32),
          pltpu.SemaphoreType.DMA,
      ],
  )
  def _pool_gather(wpix_hbm, wdep_hbm, wlid_hbm, ctx_hbm, feat_hbm,
                   outc_hbm, outl_hbm,
                   wps, wds, wls, pixr, dvr, lidr, subb, ctxb, featb, foutb, sem):
      wid = lax.axis_index("s") * 2 + lax.axis_index("c")
      vox_per_w = TOTAL // NW

      for t in range(vox_per_w // CH):
          cb = pl.multiple_of(wid * vox_per_w + t * CH, CH)
          pltpu.sync_copy(wpix_hbm.at[:, pl.ds(cb, CH)], wps)
          pltpu.sync_copy(wdep_hbm.at[:, pl.ds(cb, CH)], wds)
          pltpu.sync_copy(wlid_hbm.at[:, pl.ds(cb, CH)], wls)

          def red_body(v, _):
              def g_body(g, carry):
                  pix, dv, lid = carry
                  wp = wps[g, pl.ds(v * 16, 16)]
                  wd = wds[g, pl.ds(v * 16, 16)]
                  wl = wls[g, pl.ds(v * 16, 16)]
                  updc = wp >= 0
                  return (jnp.where(updc, wp, pix),
                          jnp.where(updc, wd, dv),
                          jnp.where(wl >= 0, wl, lid))
              pix, dv, lid = lax.fori_loop(
                  0, NW, g_body,
                  (jnp.full((16,), -1, jnp.int32), jnp.zeros((16,), jnp.float32),
                   jnp.full((16,), -1, jnp.int32)))
              lid = jnp.where(lid >= 0, lid, B * NP)
              pixr[pl.ds(v * 16, 16)] = jnp.maximum(pix, 0)
              dvr[pl.ds(v * 16, 16)] = dv
              lidr[pl.ds(v * 16, 16)] = lax.shift_right_logical(lid, 2)
              subb[pl.ds(v * 16, 16)] = (lid & 3) << 5
              return 0
          lax.fori_loop(0, CH // 16, red_body, 0)

          for s in range(CH // 128):
              pltpu.async_copy(ctx_hbm.at[pixr.at[pl.ds(s * 128, 128)]],
                               ctxb.at[pl.ds(s * 128, 128)], sem).wait()
              pltpu.async_copy(feat_hbm.at[lidr.at[pl.ds(s * 128, 128)]],
                               featb.at[pl.ds(s * 128, 128)], sem).wait()

          def w_body(vg, _):
              dv = dvr[pl.ds(vg * 16, 16)]
              sb = subb[pl.ds(vg * 16, 16)]
              for l in range(16):
                  sc = dv[l]
                  v = vg * 16 + l
                  for c in range(OUT_C // 16):
                      ctxb[v, pl.ds(c * 16, 16)] = ctxb[v, pl.ds(c * 16, 16)] * sc
                  off = pl.multiple_of(sb[l], 16)
                  foutb[v, pl.ds(0, 16)] = featb[v, pl.ds(off, 16)]
                  foutb[v, pl.ds(16, 16)] = featb[v, pl.ds(off + 16, 16)]
              return 0
          lax.fori_loop(0, CH // 16, w_body, 0)

          pltpu.sync_copy(ctxb, outc_hbm.at[pl.ds(cb, CH)])
          pltpu.sync_copy(foutb, outl_hbm.at[pl.ds(cb, CH)])

  return _pool_scatter_cam, _pool_scatter_lid, _pool_gather


def kernel(imgs, rots, trans, intrins, points, backbone_w, backbone_b, bn1_g,
           bn1_b, depth_w, depth_b, pfn_w, pfn_b, pfn_bn_g, pfn_bn_b,
           fuser_w1, fbn1_g, fbn1_b, fuser_w2, fbn2_g, fbn2_b):
    x = imgs.reshape(B * N, 3, IH, IW)
    x = jax.nn.relu(_bn2d(_conv2d(x, backbone_w, backbone_b, stride=16, pad=1), bn1_g, bn1_b))
    x = _conv2d(x, depth_w, depth_b)
    depth = jax.nn.softmax(x[:, :D], axis=1)
    ctx = x[:, D:]
    depth_flat = depth.reshape(NPRIME)
    depth_flat = jnp.concatenate(
        [depth_flat, jnp.zeros((NPRIME_PAD - NPRIME,), jnp.float32)])
    ctx_tab = ctx.transpose(0, 2, 3, 1).reshape(NPIX, OUT_C)
    ctx_tab = jnp.pad(ctx_tab, ((0, 0), (0, 128 - OUT_C)))

    fr = _make_frustum()
    pts = jnp.broadcast_to(fr[None, None], (B, N, D, FH, FW, 3))
    pts = jnp.concatenate([pts[..., :2] * pts[..., 2:3], pts[..., 2:3]], -1)
    comb = jnp.matmul(rots, jnp.linalg.inv(intrins))
    pts = jnp.einsum('bnij,bndhwj->bndhwi', comb, pts) + trans[:, :, None, None, None, :]
    bix = jnp.repeat(jnp.arange(B, dtype=jnp.int32), N * D * FH * FW)
    ranks_cam = _voxel_ranks(pts.reshape(NPRIME, 3), bix)
    ranks_cam = jnp.concatenate(
        [ranks_cam, jnp.full((NPRIME_PAD - NPRIME,), TOTAL, jnp.int32)])

    pf = points.reshape(-1, 4)
    feat = jax.nn.relu(_bn1d(pf @ pfn_w.T + pfn_b, pfn_bn_g, pfn_bn_b))
    lbix = jnp.repeat(jnp.arange(B, dtype=jnp.int32), NP)
    ranks_lid = _voxel_ranks(pf[:, :3], lbix)
    feat_pad = jnp.concatenate(
        [feat, jnp.zeros((NP_PAD - B * NP, LID_C), jnp.float32)], axis=0)
    ranks_lid_pad = jnp.concatenate(
        [ranks_lid, jnp.full((NP_PAD - B * NP,), TOTAL, jnp.int32)], axis=0)

    pool_scatter_cam, pool_scatter_lid, pool_gather = _build_sc_kernels()
    wpix, wdep = pool_scatter_cam(ranks_cam, depth_flat)
    wlid = pool_scatter_lid(ranks_lid_pad)
    _BISECT_B = False
    if _BISECT_B:
        pix = jnp.full((TOTAL,), -1, jnp.int32)
        dv = jnp.zeros((TOTAL,), jnp.float32)
        lid = jnp.full((TOTAL,), -1, jnp.int32)
        for g in range(NW):
            updc = wpix[g, :TOTAL] >= 0
            pix = jnp.where(updc, wpix[g, :TOTAL], pix)
            dv = jnp.where(updc, wdep[g, :TOTAL], dv)
            updl = wlid[g, :TOTAL] >= 0
            lid = jnp.where(updl, wlid[g, :TOTAL], lid)
        outc = ctx_tab[jnp.maximum(pix, 0), :OUT_C] * dv[:, None]
        outl = feat_pad[jnp.where(lid >= 0, lid, B * NP)]
    else:
        outc, outl = pool_gather(wpix, wdep, wlid, ctx_tab,
                                 feat_pad.reshape(NP_PAD // 4, 4 * LID_C))
        outc = outc[:, :OUT_C]

    cam_bev = outc.reshape(B, NX[1], NX[0], OUT_C).transpose(0, 3, 1, 2)
    lid_bev = outl.reshape(B, NX[1], NX[0], LID_C).transpose(0, 3, 1, 2)

    f = jnp.concatenate([cam_bev, lid_bev], axis=1)
    f = jax.nn.relu(_bn2d(_conv2d(f, fuser_w1, None, 1, 1), fbn1_g, fbn1_b))
    f = jax.nn.relu(_bn2d(_conv2d(f, fuser_w2, None, 1, 1), fbn2_g, fbn2_b))
    return f
